# baseline (speedup 1.0000x reference)
import functools

import jax
import jax.numpy as jnp
from jax import lax
from jax.experimental import pallas as pl
from jax.experimental.pallas import tpu as pltpu
from jax.experimental.pallas import tpu_sc as plsc

_N = 10000
_E = 160000
_TILES = 16
_WB = 632
_LAST = _N - (_TILES - 1) * _WB
_CH = 128

_R = 2000
_G = _N // _R

_CPT = 40
_CPC = _TILES * _CPT
_EPAD = 2 * _CPC * _CH - _E
_NPAD = _N + 240

_mesh = plsc.VectorSubcoreMesh(core_axis_name="c", subcore_axis_name="s")


def _zero_table(z_ref, table, t):
    start = pl.multiple_of(t * _WB, 8)

    @pl.when(t < _TILES - 1)
    def _():
        pltpu.sync_copy(z_ref, table.at[pl.ds(start, _WB)])

    @pl.when(t == _TILES - 1)
    def _():
        pltpu.sync_copy(z_ref.at[pl.ds(0, _LAST)], table.at[pl.ds(start, _LAST)])


def _write_table(table, out_ref, c, t):
    start = pl.multiple_of(t * _WB, 8)

    @pl.when(t < _TILES - 1)
    def _():
        pltpu.sync_copy(table.at[pl.ds(start, _WB)], out_ref.at[c, pl.ds(start, _WB)])

    @pl.when(t == _TILES - 1)
    def _():
        pltpu.sync_copy(table.at[pl.ds(start, _LAST)], out_ref.at[c, pl.ds(start, _LAST)])


def _load_tile_idx(idx3_ref, vbuf, c, t):
    row0 = pl.multiple_of(t * _CPT, 8)
    pltpu.sync_copy(idx3_ref.at[c, pl.ds(row0, _CPT)], vbuf)


def _sc_deg():
    @functools.partial(
        pl.kernel,
        out_type=jax.ShapeDtypeStruct((2, _N, 128), jnp.float32),
        mesh=_mesh,
        scratch_types=[
            pltpu.VMEM((_CPT, _CH), jnp.int32),
            pltpu.VMEM((_CH, 128), jnp.float32),
            pltpu.VMEM_SHARED((_NPAD, 128), jnp.float32),
        ],
    )
    def k(dst3_ref, oh_ref, z_ref, out_ref, didx_all, ones_v, table):
        c = lax.axis_index("c")
        t = lax.axis_index("s")
        pltpu.sync_copy(oh_ref, ones_v)
        _load_tile_idx(dst3_ref, didx_all, c, t)
        _zero_table(z_ref, table, t)
        plsc.subcore_barrier()

        def body(j, carry):
            pltpu.sync_copy(ones_v, table.at[didx_all.at[j]], add=True)
            return carry

        lax.fori_loop(0, _CPT, body, 0)
        plsc.subcore_barrier()
        _write_table(table, out_ref, c, t)

    return k


def _sc_agg(d):
    @functools.partial(
        pl.kernel,
        out_type=jax.ShapeDtypeStruct((2, _N, d), jnp.float32),
        mesh=_mesh,
        scratch_types=[
            pltpu.VMEM((_CPT, _CH), jnp.int32),
            pltpu.VMEM((_CPT, _CH), jnp.int32),
            pltpu.VMEM((2, _CH, d), jnp.float32),
            pltpu.VMEM_SHARED((_NPAD, d), jnp.float32),
            pltpu.SemaphoreType.DMA((2,)),
        ],
    )
    def k(src3_ref, dst3_ref, s_ref, z_ref, out_ref, sidx_all, didx_all, rows2, table, sem):
        c = lax.axis_index("c")
        t = lax.axis_index("s")
        _load_tile_idx(src3_ref, sidx_all, c, t)
        _load_tile_idx(dst3_ref, didx_all, c, t)
        _zero_table(z_ref, table, t)
        plsc.subcore_barrier()

        def gather(j):
            slot = lax.rem(j, 2)
            return pltpu.make_async_copy(
                s_ref.at[sidx_all.at[j]], rows2.at[slot], sem.at[slot]
            )

        def body(j, carry):
            g = gather(j)
            g.start()
            g.wait()
            pltpu.sync_copy(rows2.at[lax.rem(j, 2)], table.at[didx_all.at[j]], add=True)
            return carry

        lax.fori_loop(0, _CPT, body, 0)
        plsc.subcore_barrier()
        _write_table(table, out_ref, c, t)

    return k


def _tc_pre1(x, W, degtab):
    din, dout = W.shape

    def body(x_ref, w_ref, dg_ref, dinv_ref, s_ref):
        dinv = lax.rsqrt(1.0 + dg_ref[0, :, 0:1] + dg_ref[1, :, 0:1])
        h = jnp.dot(x_ref[...], w_ref[...], preferred_element_type=jnp.float32)
        dinv_ref[...] = dinv
        s_ref[...] = dinv * h

    return pl.pallas_call(
        body,
        grid=(_G,),
        in_specs=[
            pl.BlockSpec((_R, din), lambda i: (i, 0)),
            pl.BlockSpec((din, dout), lambda i: (0, 0)),
            pl.BlockSpec((2, _R, 128), lambda i: (0, i, 0)),
        ],
        out_specs=[
            pl.BlockSpec((_R, 1), lambda i: (i, 0)),
            pl.BlockSpec((_R, dout), lambda i: (i, 0)),
        ],
        out_shape=[
            jax.ShapeDtypeStruct((_N, 1), jnp.float32),
            jax.ShapeDtypeStruct((_N, dout), jnp.float32),
        ],
    )(x, W, degtab)


def _tc_mid(p, s_prev, dinv, b, W, mode, stack_out=False, take=None, pad_to=None):
    if W is None:
        din = dout = b.shape[1]
    else:
        din, dout = W.shape
    sin = s_prev.shape[1]

    def body(*refs):
        if W is None:
            p_ref, s_ref, dv_ref, b_ref, o_ref = refs
        else:
            p_ref, s_ref, dv_ref, b_ref, w_ref, o_ref = refs
        dv = dv_ref[...]
        agg = p_ref[0] + p_ref[1] + s_ref[...]
        if take is not None:
            agg = agg[:, :take]
        h = dv * agg + b_ref[...]
        if mode == "relu":
            h = jnp.maximum(h, 0.0)
        elif mode == "norm":
            n = jnp.sqrt(jnp.sum(h * h, axis=1, keepdims=True))
            h = h / jnp.maximum(n, 1e-12)
        if W is None:
            s_next = dv * h
        else:
            s_next = dv * jnp.dot(h, w_ref[...], preferred_element_type=jnp.float32)
        if stack_out:
            o_ref[0] = s_next[:, : dout // 2]
            o_ref[1] = s_next[:, dout // 2 :]
        elif pad_to is not None:
            o_ref[...] = jnp.concatenate(
                [s_next, jnp.zeros((_R, pad_to - dout), jnp.float32)], axis=1
            )
        else:
            o_ref[...] = s_next

    if stack_out:
        out_spec = pl.BlockSpec((2, _R, dout // 2), lambda i: (0, i, 0))
        out_shape = jax.ShapeDtypeStruct((2, _N, dout // 2), jnp.float32)
    else:
        ocols = pad_to if pad_to is not None else dout
        out_spec = pl.BlockSpec((_R, ocols), lambda i: (i, 0))
        out_shape = jax.ShapeDtypeStruct((_N, ocols), jnp.float32)

    in_specs = [
        pl.BlockSpec((2, _R, sin), lambda i: (0, i, 0)),
        pl.BlockSpec((_R, sin), lambda i: (i, 0)),
        pl.BlockSpec((_R, 1), lambda i: (i, 0)),
        pl.BlockSpec((1, din), lambda i: (0, 0)),
    ]
    args = [p, s_prev, dinv, b]
    if W is not None:
        in_specs.append(pl.BlockSpec((din, dout), lambda i: (0, 0)))
        args.append(W)

    return pl.pallas_call(
        body,
        grid=(_G,),
        in_specs=in_specs,
        out_specs=out_spec,
        out_shape=out_shape,
    )(*args)


def _tc_post(p4, h3t, dinv, b, W):
    din, dout = W.shape

    def body(p_ref, s_ref, dv_ref, b_ref, w_ref, o_ref):
        tot = p_ref[0] + p_ref[1] + s_ref[...]
        mm = jnp.dot(tot, w_ref[...], preferred_element_type=jnp.float32)
        o_ref[...] = dv_ref[...] * mm + b_ref[...]

    return pl.pallas_call(
        body,
        grid=(_G,),
        in_specs=[
            pl.BlockSpec((2, _R, din), lambda i: (0, i, 0)),
            pl.BlockSpec((_R, din), lambda i: (i, 0)),
            pl.BlockSpec((_R, 1), lambda i: (i, 0)),
            pl.BlockSpec((1, dout), lambda i: (0, 0)),
            pl.BlockSpec((din, dout), lambda i: (0, 0)),
        ],
        out_specs=pl.BlockSpec((_R, dout), lambda i: (i, 0)),
        out_shape=jax.ShapeDtypeStruct((_N, dout), jnp.float32),
    )(p4, h3t, dinv, b, W)


def kernel(x, edge_index, W_e1, b_e1, W_e2, b_e2, W_d1, b_d1, W_d2, b_d2):
    pad_src = jnp.zeros((_EPAD,), edge_index.dtype)
    pad_dst = _N + jnp.arange(_EPAD, dtype=edge_index.dtype) % (_NPAD - _N)
    src3 = jnp.concatenate([edge_index[0], pad_src]).reshape(2, _CPC, _CH)
    dst3 = jnp.concatenate([edge_index[1], pad_dst]).reshape(2, _CPC, _CH)
    onehot = jnp.zeros((_CH, 128), jnp.float32).at[:, 0].set(1.0)
    z128 = jnp.zeros((_WB, 128), jnp.float32)

    degtab = _sc_deg()(dst3, onehot, z128)
    dinv, s1 = _tc_pre1(x, W_e1, degtab)
    p1 = _sc_agg(128)(src3, dst3, s1, z128)
    s2 = _tc_mid(p1, s1, dinv, b_e1.reshape(1, -1), W_e2, "relu", pad_to=128)
    p2 = _sc_agg(128)(src3, dst3, s2, z128)
    s3 = _tc_mid(p2, s2, dinv, b_e2.reshape(1, -1), W_d1, "norm", take=64)
    p3 = _sc_agg(128)(src3, dst3, s3, z128)
    h3t = _tc_mid(p3, s3, dinv, b_d1.reshape(1, -1), None, "relu")
    p4 = _sc_agg(128)(src3, dst3, h3t, z128)
    x_hat = _tc_post(p4, h3t, dinv, b_d2.reshape(1, -1), W_d2)
    return x_hat


# speedup vs baseline: 1.0080x; 1.0080x over previous
import functools

import jax
import jax.numpy as jnp
from jax import lax
from jax.experimental import pallas as pl
from jax.experimental.pallas import tpu as pltpu
from jax.experimental.pallas import tpu_sc as plsc

_N = 10000
_E = 160000
_TILES = 16
_WB = 632
_LAST = _N - (_TILES - 1) * _WB
_CH = 128

_R = 2000
_G = _N // _R

_CPT = 40
_CPC = _TILES * _CPT
_EPAD = 2 * _CPC * _CH - _E
_NPAD = _N + 240

_mesh = plsc.VectorSubcoreMesh(core_axis_name="c", subcore_axis_name="s")


def _zero_table(z_ref, table, t):
    start = pl.multiple_of(t * _WB, 8)

    @pl.when(t < _TILES - 1)
    def _():
        pltpu.sync_copy(z_ref, table.at[pl.ds(start, _WB)])

    @pl.when(t == _TILES - 1)
    def _():
        pltpu.sync_copy(z_ref.at[pl.ds(0, _LAST)], table.at[pl.ds(start, _LAST)])


def _write_table(table, out_ref, c, t):
    start = pl.multiple_of(t * _WB, 8)

    @pl.when(t < _TILES - 1)
    def _():
        pltpu.sync_copy(table.at[pl.ds(start, _WB)], out_ref.at[c, pl.ds(start, _WB)])

    @pl.when(t == _TILES - 1)
    def _():
        pltpu.sync_copy(table.at[pl.ds(start, _LAST)], out_ref.at[c, pl.ds(start, _LAST)])


def _sc_deg():
    @functools.partial(
        pl.kernel,
        out_type=jax.ShapeDtypeStruct((2, _N, 128), jnp.float32),
        mesh=_mesh,
        scratch_types=[
            pltpu.VMEM((8, _CH), jnp.int32),
            pltpu.VMEM((_CH, 128), jnp.float32),
            pltpu.VMEM_SHARED((_NPAD, 128), jnp.float32),
        ],
    )
    def k(dst2_ref, oh_ref, z_ref, out_ref, didx8, ones_v, table):
        c = lax.axis_index("c")
        t = lax.axis_index("s")
        pltpu.sync_copy(oh_ref, ones_v)
        _zero_table(z_ref, table, t)
        plsc.subcore_barrier()

        def body(i, carry):
            row0 = pl.multiple_of((c * _CPC + t * _CPT) + 8 * i, 8)
            pltpu.sync_copy(dst2_ref.at[pl.ds(row0, 8)], didx8)
            for k8 in range(8):
                pltpu.sync_copy(ones_v, table.at[didx8.at[k8]], add=True)
            return carry

        lax.fori_loop(0, _CPT // 8, body, 0)
        plsc.subcore_barrier()
        _write_table(table, out_ref, c, t)

    return k


def _sc_agg(d):
    @functools.partial(
        pl.kernel,
        out_type=jax.ShapeDtypeStruct((2, _N, d), jnp.float32),
        mesh=_mesh,
        scratch_types=[
            pltpu.VMEM((8, _CH), jnp.int32),
            pltpu.VMEM((8, _CH), jnp.int32),
            pltpu.VMEM((_CH, d), jnp.float32),
            pltpu.VMEM((_CH, d), jnp.float32),
            pltpu.VMEM_SHARED((_NPAD, d), jnp.float32),
            pltpu.SemaphoreType.DMA,
            pltpu.SemaphoreType.DMA,
        ],
    )
    def k(src2_ref, dst2_ref, s_ref, z_ref, out_ref, sidx8, didx8, ra, rb, table, ma, mb):
        c = lax.axis_index("c")
        t = lax.axis_index("s")
        _zero_table(z_ref, table, t)
        plsc.subcore_barrier()

        def gather(k8, rows, sem):
            return pltpu.make_async_copy(s_ref.at[sidx8.at[k8]], rows, sem)

        def body(i, carry):
            row0 = pl.multiple_of((c * _CPC + t * _CPT) + 8 * i, 8)
            pltpu.sync_copy(src2_ref.at[pl.ds(row0, 8)], sidx8)
            pltpu.sync_copy(dst2_ref.at[pl.ds(row0, 8)], didx8)
            bufs = [(ra, ma), (rb, mb)]
            gather(0, *bufs[0]).start()
            for k8 in range(8):
                rows, sem = bufs[k8 % 2]
                if k8 + 1 < 8:
                    gather(k8 + 1, *bufs[(k8 + 1) % 2]).start()
                gather(k8, rows, sem).wait()
                pltpu.sync_copy(rows, table.at[didx8.at[k8]], add=True)
            return carry

        lax.fori_loop(0, _CPT // 8, body, 0)
        plsc.subcore_barrier()
        _write_table(table, out_ref, c, t)

    return k


def _tc_pre1(x, W, degtab):
    din, dout = W.shape

    def body(x_ref, w_ref, dg_ref, dinv_ref, s_ref):
        dinv = lax.rsqrt(1.0 + dg_ref[0, :, 0:1] + dg_ref[1, :, 0:1])
        h = jnp.dot(x_ref[...], w_ref[...], preferred_element_type=jnp.float32)
        dinv_ref[...] = dinv
        s_ref[...] = dinv * h

    return pl.pallas_call(
        body,
        grid=(_G,),
        in_specs=[
            pl.BlockSpec((_R, din), lambda i: (i, 0)),
            pl.BlockSpec((din, dout), lambda i: (0, 0)),
            pl.BlockSpec((2, _R, 128), lambda i: (0, i, 0)),
        ],
        out_specs=[
            pl.BlockSpec((_R, 1), lambda i: (i, 0)),
            pl.BlockSpec((_R, dout), lambda i: (i, 0)),
        ],
        out_shape=[
            jax.ShapeDtypeStruct((_N, 1), jnp.float32),
            jax.ShapeDtypeStruct((_N, dout), jnp.float32),
        ],
    )(x, W, degtab)


def _tc_mid(p, s_prev, dinv, b, W, mode, stack_out=False, take=None, pad_to=None):
    if W is None:
        din = dout = b.shape[1]
    else:
        din, dout = W.shape
    sin = s_prev.shape[1]

    def body(*refs):
        if W is None:
            p_ref, s_ref, dv_ref, b_ref, o_ref = refs
        else:
            p_ref, s_ref, dv_ref, b_ref, w_ref, o_ref = refs
        dv = dv_ref[...]
        agg = p_ref[0] + p_ref[1] + s_ref[...]
        if take is not None:
            agg = agg[:, :take]
        h = dv * agg + b_ref[...]
        if mode == "relu":
            h = jnp.maximum(h, 0.0)
        elif mode == "norm":
            n = jnp.sqrt(jnp.sum(h * h, axis=1, keepdims=True))
            h = h / jnp.maximum(n, 1e-12)
        if W is None:
            s_next = dv * h
        else:
            s_next = dv * jnp.dot(h, w_ref[...], preferred_element_type=jnp.float32)
        if stack_out:
            o_ref[0] = s_next[:, : dout // 2]
            o_ref[1] = s_next[:, dout // 2 :]
        elif pad_to is not None:
            o_ref[...] = jnp.concatenate(
                [s_next, jnp.zeros((_R, pad_to - dout), jnp.float32)], axis=1
            )
        else:
            o_ref[...] = s_next

    if stack_out:
        out_spec = pl.BlockSpec((2, _R, dout // 2), lambda i: (0, i, 0))
        out_shape = jax.ShapeDtypeStruct((2, _N, dout // 2), jnp.float32)
    else:
        ocols = pad_to if pad_to is not None else dout
        out_spec = pl.BlockSpec((_R, ocols), lambda i: (i, 0))
        out_shape = jax.ShapeDtypeStruct((_N, ocols), jnp.float32)

    in_specs = [
        pl.BlockSpec((2, _R, sin), lambda i: (0, i, 0)),
        pl.BlockSpec((_R, sin), lambda i: (i, 0)),
        pl.BlockSpec((_R, 1), lambda i: (i, 0)),
        pl.BlockSpec((1, din), lambda i: (0, 0)),
    ]
    args = [p, s_prev, dinv, b]
    if W is not None:
        in_specs.append(pl.BlockSpec((din, dout), lambda i: (0, 0)))
        args.append(W)

    return pl.pallas_call(
        body,
        grid=(_G,),
        in_specs=in_specs,
        out_specs=out_spec,
        out_shape=out_shape,
    )(*args)


def _tc_post(p4, h3t, dinv, b, W):
    din, dout = W.shape

    def body(p_ref, s_ref, dv_ref, b_ref, w_ref, o_ref):
        tot = p_ref[0] + p_ref[1] + s_ref[...]
        mm = jnp.dot(tot, w_ref[...], preferred_element_type=jnp.float32)
        o_ref[...] = dv_ref[...] * mm + b_ref[...]

    return pl.pallas_call(
        body,
        grid=(_G,),
        in_specs=[
            pl.BlockSpec((2, _R, din), lambda i: (0, i, 0)),
            pl.BlockSpec((_R, din), lambda i: (i, 0)),
            pl.BlockSpec((_R, 1), lambda i: (i, 0)),
            pl.BlockSpec((1, dout), lambda i: (0, 0)),
            pl.BlockSpec((din, dout), lambda i: (0, 0)),
        ],
        out_specs=pl.BlockSpec((_R, dout), lambda i: (i, 0)),
        out_shape=jax.ShapeDtypeStruct((_N, dout), jnp.float32),
    )(p4, h3t, dinv, b, W)


def kernel(x, edge_index, W_e1, b_e1, W_e2, b_e2, W_d1, b_d1, W_d2, b_d2):
    pad_src = jnp.zeros((_EPAD,), edge_index.dtype)
    pad_dst = _N + jnp.arange(_EPAD, dtype=edge_index.dtype) % (_NPAD - _N)
    src3 = jnp.concatenate([edge_index[0], pad_src]).reshape(2 * _CPC, _CH)
    dst3 = jnp.concatenate([edge_index[1], pad_dst]).reshape(2 * _CPC, _CH)
    onehot = jnp.zeros((_CH, 128), jnp.float32).at[:, 0].set(1.0)
    z128 = jnp.zeros((_WB, 128), jnp.float32)

    degtab = _sc_deg()(dst3, onehot, z128)
    dinv, s1 = _tc_pre1(x, W_e1, degtab)
    p1 = _sc_agg(128)(src3, dst3, s1, z128)
    s2 = _tc_mid(p1, s1, dinv, b_e1.reshape(1, -1), W_e2, "relu", pad_to=128)
    p2 = _sc_agg(128)(src3, dst3, s2, z128)
    s3 = _tc_mid(p2, s2, dinv, b_e2.reshape(1, -1), W_d1, "norm", take=64)
    p3 = _sc_agg(128)(src3, dst3, s3, z128)
    h3t = _tc_mid(p3, s3, dinv, b_d1.reshape(1, -1), None, "relu")
    p4 = _sc_agg(128)(src3, dst3, h3t, z128)
    x_hat = _tc_post(p4, h3t, dinv, b_d2.reshape(1, -1), W_d2)
    return x_hat


# speedup vs baseline: 1.0726x; 1.0640x over previous
import functools

import jax
import jax.numpy as jnp
from jax import lax
from jax.experimental import pallas as pl
from jax.experimental.pallas import tpu as pltpu
from jax.experimental.pallas import tpu_sc as plsc

_N = 10000
_E = 160000
_TILES = 16
_WB = 632
_LAST = _N - (_TILES - 1) * _WB
_CH = 128

_R = 2000
_G = _N // _R

_CPT = 40
_CPC = _TILES * _CPT
_EPAD = 2 * _CPC * _CH - _E
_NPAD = _N + 240

_mesh = plsc.VectorSubcoreMesh(core_axis_name="c", subcore_axis_name="s")


def _zero_table(z_ref, table, t):
    start = pl.multiple_of(t * _WB, 8)

    @pl.when(t < _TILES - 1)
    def _():
        pltpu.sync_copy(z_ref, table.at[pl.ds(start, _WB)])

    @pl.when(t == _TILES - 1)
    def _():
        pltpu.sync_copy(z_ref.at[pl.ds(0, _LAST)], table.at[pl.ds(start, _LAST)])


def _write_table(table, out_ref, c, t):
    start = pl.multiple_of(t * _WB, 8)

    @pl.when(t < _TILES - 1)
    def _():
        pltpu.sync_copy(table.at[pl.ds(start, _WB)], out_ref.at[c, pl.ds(start, _WB)])

    @pl.when(t == _TILES - 1)
    def _():
        pltpu.sync_copy(table.at[pl.ds(start, _LAST)], out_ref.at[c, pl.ds(start, _LAST)])


def _sc_deg():
    @functools.partial(
        pl.kernel,
        out_type=jax.ShapeDtypeStruct((2, _N, 128), jnp.float32),
        mesh=_mesh,
        scratch_types=[
            pltpu.VMEM((8, _CH), jnp.int32),
            pltpu.VMEM((_CH, 128), jnp.float32),
            pltpu.VMEM_SHARED((_NPAD, 128), jnp.float32),
        ],
    )
    def k(dst2_ref, oh_ref, z_ref, out_ref, didx8, ones_v, table):
        c = lax.axis_index("c")
        t = lax.axis_index("s")
        pltpu.sync_copy(oh_ref, ones_v)
        _zero_table(z_ref, table, t)
        plsc.subcore_barrier()

        def body(i, carry):
            row0 = pl.multiple_of((c * _CPC + t * _CPT) + 8 * i, 8)
            pltpu.sync_copy(dst2_ref.at[pl.ds(row0, 8)], didx8)
            for k8 in range(8):
                pltpu.sync_copy(ones_v, table.at[didx8.at[k8]], add=True)
            return carry

        lax.fori_loop(0, _CPT // 8, body, 0)
        plsc.subcore_barrier()
        _write_table(table, out_ref, c, t)

    return k


def _sc_agg(d):
    @functools.partial(
        pl.kernel,
        out_type=jax.ShapeDtypeStruct((2, _N, d), jnp.float32),
        mesh=_mesh,
        scratch_types=[
            pltpu.VMEM((_CH,), jnp.int32),
            pltpu.VMEM((_CH,), jnp.int32),
            pltpu.VMEM((_CH,), jnp.int32),
            pltpu.VMEM((_CH,), jnp.int32),
            pltpu.VMEM((_CH, d), jnp.float32),
            pltpu.VMEM((_CH, d), jnp.float32),
            pltpu.VMEM_SHARED((_NPAD, d), jnp.float32),
            pltpu.SemaphoreType.DMA,
            pltpu.SemaphoreType.DMA,
        ],
    )
    def k(src_ref, dst_ref, s_ref, z_ref, out_ref, sa, da, sb, db, ra, rb, table, ma, mb):
        c = lax.axis_index("c")
        t = lax.axis_index("s")
        _zero_table(z_ref, table, t)
        plsc.subcore_barrier()
        base0 = (c * _CPC + t * _CPT) * _CH

        def body(i, carry):
            b0 = pl.multiple_of(base0 + (2 * i) * _CH, _CH)
            b1 = pl.multiple_of(base0 + (2 * i + 1) * _CH, _CH)
            pltpu.sync_copy(src_ref.at[pl.ds(b0, _CH)], sa)
            pltpu.sync_copy(dst_ref.at[pl.ds(b0, _CH)], da)
            h0 = pltpu.async_copy(s_ref.at[sa], ra, ma)
            pltpu.sync_copy(src_ref.at[pl.ds(b1, _CH)], sb)
            pltpu.sync_copy(dst_ref.at[pl.ds(b1, _CH)], db)
            h1 = pltpu.async_copy(s_ref.at[sb], rb, mb)
            h0.wait()
            pltpu.sync_copy(ra, table.at[da], add=True)
            h1.wait()
            pltpu.sync_copy(rb, table.at[db], add=True)
            return carry

        lax.fori_loop(0, _CPT // 2, body, 0)
        plsc.subcore_barrier()
        _write_table(table, out_ref, c, t)

    return k


def _tc_pre1(x, W, degtab):
    din, dout = W.shape

    def body(x_ref, w_ref, dg_ref, dinv_ref, s_ref):
        dinv = lax.rsqrt(1.0 + dg_ref[0, :, 0:1] + dg_ref[1, :, 0:1])
        h = jnp.dot(x_ref[...], w_ref[...], preferred_element_type=jnp.float32)
        dinv_ref[...] = dinv
        s_ref[...] = dinv * h

    return pl.pallas_call(
        body,
        grid=(_G,),
        in_specs=[
            pl.BlockSpec((_R, din), lambda i: (i, 0)),
            pl.BlockSpec((din, dout), lambda i: (0, 0)),
            pl.BlockSpec((2, _R, 128), lambda i: (0, i, 0)),
        ],
        out_specs=[
            pl.BlockSpec((_R, 1), lambda i: (i, 0)),
            pl.BlockSpec((_R, dout), lambda i: (i, 0)),
        ],
        out_shape=[
            jax.ShapeDtypeStruct((_N, 1), jnp.float32),
            jax.ShapeDtypeStruct((_N, dout), jnp.float32),
        ],
    )(x, W, degtab)


def _tc_mid(p, s_prev, dinv, b, W, mode, stack_out=False, take=None, pad_to=None):
    if W is None:
        din = dout = b.shape[1]
    else:
        din, dout = W.shape
    sin = s_prev.shape[1]

    def body(*refs):
        if W is None:
            p_ref, s_ref, dv_ref, b_ref, o_ref = refs
        else:
            p_ref, s_ref, dv_ref, b_ref, w_ref, o_ref = refs
        dv = dv_ref[...]
        agg = p_ref[0] + p_ref[1] + s_ref[...]
        if take is not None:
            agg = agg[:, :take]
        h = dv * agg + b_ref[...]
        if mode == "relu":
            h = jnp.maximum(h, 0.0)
        elif mode == "norm":
            n = jnp.sqrt(jnp.sum(h * h, axis=1, keepdims=True))
            h = h / jnp.maximum(n, 1e-12)
        if W is None:
            s_next = dv * h
        else:
            s_next = dv * jnp.dot(h, w_ref[...], preferred_element_type=jnp.float32)
        if stack_out:
            o_ref[0] = s_next[:, : dout // 2]
            o_ref[1] = s_next[:, dout // 2 :]
        elif pad_to is not None:
            o_ref[...] = jnp.concatenate(
                [s_next, jnp.zeros((_R, pad_to - dout), jnp.float32)], axis=1
            )
        else:
            o_ref[...] = s_next

    if stack_out:
        out_spec = pl.BlockSpec((2, _R, dout // 2), lambda i: (0, i, 0))
        out_shape = jax.ShapeDtypeStruct((2, _N, dout // 2), jnp.float32)
    else:
        ocols = pad_to if pad_to is not None else dout
        out_spec = pl.BlockSpec((_R, ocols), lambda i: (i, 0))
        out_shape = jax.ShapeDtypeStruct((_N, ocols), jnp.float32)

    in_specs = [
        pl.BlockSpec((2, _R, sin), lambda i: (0, i, 0)),
        pl.BlockSpec((_R, sin), lambda i: (i, 0)),
        pl.BlockSpec((_R, 1), lambda i: (i, 0)),
        pl.BlockSpec((1, din), lambda i: (0, 0)),
    ]
    args = [p, s_prev, dinv, b]
    if W is not None:
        in_specs.append(pl.BlockSpec((din, dout), lambda i: (0, 0)))
        args.append(W)

    return pl.pallas_call(
        body,
        grid=(_G,),
        in_specs=in_specs,
        out_specs=out_spec,
        out_shape=out_shape,
    )(*args)


def _tc_post(p4, h3t, dinv, b, W):
    din, dout = W.shape

    def body(p_ref, s_ref, dv_ref, b_ref, w_ref, o_ref):
        tot = p_ref[0] + p_ref[1] + s_ref[...]
        mm = jnp.dot(tot, w_ref[...], preferred_element_type=jnp.float32)
        o_ref[...] = dv_ref[...] * mm + b_ref[...]

    return pl.pallas_call(
        body,
        grid=(_G,),
        in_specs=[
            pl.BlockSpec((2, _R, din), lambda i: (0, i, 0)),
            pl.BlockSpec((_R, din), lambda i: (i, 0)),
            pl.BlockSpec((_R, 1), lambda i: (i, 0)),
            pl.BlockSpec((1, dout), lambda i: (0, 0)),
            pl.BlockSpec((din, dout), lambda i: (0, 0)),
        ],
        out_specs=pl.BlockSpec((_R, dout), lambda i: (i, 0)),
        out_shape=jax.ShapeDtypeStruct((_N, dout), jnp.float32),
    )(p4, h3t, dinv, b, W)


def kernel(x, edge_index, W_e1, b_e1, W_e2, b_e2, W_d1, b_d1, W_d2, b_d2):
    pad_src = jnp.zeros((_EPAD,), edge_index.dtype)
    pad_dst = _N + jnp.arange(_EPAD, dtype=edge_index.dtype) % (_NPAD - _N)
    srcf = jnp.concatenate([edge_index[0], pad_src])
    dstf = jnp.concatenate([edge_index[1], pad_dst])
    dst2d = dstf.reshape(2 * _CPC, _CH)
    onehot = jnp.zeros((_CH, 128), jnp.float32).at[:, 0].set(1.0)
    z128 = jnp.zeros((_WB, 128), jnp.float32)

    degtab = _sc_deg()(dst2d, onehot, z128)
    dinv, s1 = _tc_pre1(x, W_e1, degtab)
    p1 = _sc_agg(128)(srcf, dstf, s1, z128)
    s2 = _tc_mid(p1, s1, dinv, b_e1.reshape(1, -1), W_e2, "relu", pad_to=128)
    p2 = _sc_agg(128)(srcf, dstf, s2, z128)
    s3 = _tc_mid(p2, s2, dinv, b_e2.reshape(1, -1), W_d1, "norm", take=64)
    p3 = _sc_agg(128)(srcf, dstf, s3, z128)
    h3t = _tc_mid(p3, s3, dinv, b_d1.reshape(1, -1), None, "relu")
    p4 = _sc_agg(128)(srcf, dstf, h3t, z128)
    x_hat = _tc_post(p4, h3t, dinv, b_d2.reshape(1, -1), W_d2)
    return x_hat


# speedup vs baseline: 2.4562x; 2.2900x over previous
import functools

import jax
import jax.numpy as jnp
from jax import lax
from jax.experimental import pallas as pl
from jax.experimental.pallas import tpu as pltpu
from jax.experimental.pallas import tpu_sc as plsc

_N = 10000
_E = 160000
_TILES = 16
_WB = 640
_CH = 128

_R = 2000
_G = _N // _R

_CPT = 40
_CPC = _TILES * _CPT
_EPAD = 2 * _CPC * _CH - _E
_NPAD = _N + 240

_mesh = plsc.VectorSubcoreMesh(core_axis_name="c", subcore_axis_name="s")


def _zero_table(z_ref, table, t):
    start = pl.multiple_of(t * _WB, 8)
    pltpu.sync_copy(z_ref, table.at[pl.ds(start, _WB)])


def _write_table(table, out_ref, c, t):
    start = pl.multiple_of(t * _WB, 8)

    @pl.when(t < _TILES - 1)
    def _():
        pltpu.sync_copy(table.at[pl.ds(start, _WB)], out_ref.at[c, pl.ds(start, _WB)])

    @pl.when(t == _TILES - 1)
    def _():
        last = _N - (_TILES - 1) * _WB
        pltpu.sync_copy(table.at[pl.ds(start, last)], out_ref.at[c, pl.ds(start, last)])


def _sc_deg():
    @functools.partial(
        pl.kernel,
        out_type=jax.ShapeDtypeStruct((2, _N, 128), jnp.float32),
        mesh=_mesh,
        scratch_types=[
            pltpu.VMEM((8, _CH), jnp.int32),
            pltpu.VMEM((_CH, 128), jnp.float32),
            pltpu.VMEM_SHARED((_NPAD, 128), jnp.float32),
        ],
    )
    def k(dst2_ref, oh_ref, z_ref, out_ref, didx8, ones_v, table):
        c = lax.axis_index("c")
        t = lax.axis_index("s")
        pltpu.sync_copy(oh_ref, ones_v)
        _zero_table(z_ref, table, t)
        plsc.subcore_barrier()

        def body(i, carry):
            row0 = pl.multiple_of((c * _CPC + t * _CPT) + 8 * i, 8)
            pltpu.sync_copy(dst2_ref.at[pl.ds(row0, 8)], didx8)
            for k8 in range(8):
                pltpu.sync_copy(ones_v, table.at[didx8.at[k8]], add=True)
            return carry

        lax.fori_loop(0, _CPT // 8, body, 0)
        plsc.subcore_barrier()
        _write_table(table, out_ref, c, t)

    return k


def _sc_agg(d):
    @functools.partial(
        pl.kernel,
        out_type=jax.ShapeDtypeStruct((2, _N, d), jnp.float32),
        mesh=_mesh,
        scratch_types=[
            pltpu.VMEM((_CH,), jnp.int32),
            pltpu.VMEM((_CH,), jnp.int32),
            pltpu.VMEM((_CH,), jnp.int32),
            pltpu.VMEM((_CH,), jnp.int32),
            pltpu.VMEM((_CH, d), jnp.float32),
            pltpu.VMEM((_CH, d), jnp.float32),
            pltpu.VMEM_SHARED((_NPAD, d), jnp.float32),
            pltpu.SemaphoreType.DMA,
            pltpu.SemaphoreType.DMA,
        ],
    )
    def k(src_ref, dst_ref, s_ref, z_ref, out_ref, sa, da, sb, db, ra, rb, table, ma, mb):
        c = lax.axis_index("c")
        t = lax.axis_index("s")
        _zero_table(z_ref, table, t)
        plsc.subcore_barrier()
        base0 = (c * _CPC + t * _CPT) * _CH

        def body(i, carry):
            b0 = pl.multiple_of(base0 + (2 * i) * _CH, _CH)
            b1 = pl.multiple_of(base0 + (2 * i + 1) * _CH, _CH)
            pltpu.sync_copy(src_ref.at[pl.ds(b0, _CH)], sa)
            pltpu.sync_copy(dst_ref.at[pl.ds(b0, _CH)], da)
            h0 = pltpu.async_copy(s_ref.at[sa], ra, ma)
            pltpu.sync_copy(src_ref.at[pl.ds(b1, _CH)], sb)
            pltpu.sync_copy(dst_ref.at[pl.ds(b1, _CH)], db)
            h1 = pltpu.async_copy(s_ref.at[sb], rb, mb)
            h0.wait()
            pltpu.sync_copy(ra, table.at[da], add=True)
            h1.wait()
            pltpu.sync_copy(rb, table.at[db], add=True)
            return carry

        lax.fori_loop(0, _CPT // 2, body, 0)
        plsc.subcore_barrier()
        _write_table(table, out_ref, c, t)

    return k


def _tc_pre1(x, W, degtab):
    din, dout = W.shape

    def body(x_ref, w_ref, dg_ref, dinv_ref, s_ref):
        dinv = lax.rsqrt(1.0 + dg_ref[0, :, 0:1] + dg_ref[1, :, 0:1])
        h = jnp.dot(x_ref[...], w_ref[...], preferred_element_type=jnp.float32)
        dinv_ref[...] = dinv
        s_ref[...] = dinv * h

    return pl.pallas_call(
        body,
        grid=(_G,),
        in_specs=[
            pl.BlockSpec((_R, din), lambda i: (i, 0)),
            pl.BlockSpec((din, dout), lambda i: (0, 0)),
            pl.BlockSpec((2, _R, 128), lambda i: (0, i, 0)),
        ],
        out_specs=[
            pl.BlockSpec((_R, 1), lambda i: (i, 0)),
            pl.BlockSpec((_R, dout), lambda i: (i, 0)),
        ],
        out_shape=[
            jax.ShapeDtypeStruct((_N, 1), jnp.float32),
            jax.ShapeDtypeStruct((_N, dout), jnp.float32),
        ],
    )(x, W, degtab)


def _tc_mid(p, s_prev, dinv, b, W, mode, stack_out=False, take=None, pad_to=None):
    if W is None:
        din = dout = b.shape[1]
    else:
        din, dout = W.shape
    sin = s_prev.shape[1]

    def body(*refs):
        if W is None:
            p_ref, s_ref, dv_ref, b_ref, o_ref = refs
        else:
            p_ref, s_ref, dv_ref, b_ref, w_ref, o_ref = refs
        dv = dv_ref[...]
        agg = p_ref[0] + p_ref[1] + s_ref[...]
        if take is not None:
            agg = agg[:, :take]
        h = dv * agg + b_ref[...]
        if mode == "relu":
            h = jnp.maximum(h, 0.0)
        elif mode == "norm":
            n = jnp.sqrt(jnp.sum(h * h, axis=1, keepdims=True))
            h = h / jnp.maximum(n, 1e-12)
        if W is None:
            s_next = dv * h
        else:
            s_next = dv * jnp.dot(h, w_ref[...], preferred_element_type=jnp.float32)
        if stack_out:
            o_ref[0] = s_next[:, : dout // 2]
            o_ref[1] = s_next[:, dout // 2 :]
        elif pad_to is not None:
            o_ref[...] = jnp.concatenate(
                [s_next, jnp.zeros((_R, pad_to - dout), jnp.float32)], axis=1
            )
        else:
            o_ref[...] = s_next

    if stack_out:
        out_spec = pl.BlockSpec((2, _R, dout // 2), lambda i: (0, i, 0))
        out_shape = jax.ShapeDtypeStruct((2, _N, dout // 2), jnp.float32)
    else:
        ocols = pad_to if pad_to is not None else dout
        out_spec = pl.BlockSpec((_R, ocols), lambda i: (i, 0))
        out_shape = jax.ShapeDtypeStruct((_N, ocols), jnp.float32)

    in_specs = [
        pl.BlockSpec((2, _R, sin), lambda i: (0, i, 0)),
        pl.BlockSpec((_R, sin), lambda i: (i, 0)),
        pl.BlockSpec((_R, 1), lambda i: (i, 0)),
        pl.BlockSpec((1, din), lambda i: (0, 0)),
    ]
    args = [p, s_prev, dinv, b]
    if W is not None:
        in_specs.append(pl.BlockSpec((din, dout), lambda i: (0, 0)))
        args.append(W)

    return pl.pallas_call(
        body,
        grid=(_G,),
        in_specs=in_specs,
        out_specs=out_spec,
        out_shape=out_shape,
    )(*args)


def _tc_post(p4, h3t, dinv, b, W):
    din, dout = W.shape

    def body(p_ref, s_ref, dv_ref, b_ref, w_ref, o_ref):
        tot = p_ref[0] + p_ref[1] + s_ref[...]
        mm = jnp.dot(tot, w_ref[...], preferred_element_type=jnp.float32)
        o_ref[...] = dv_ref[...] * mm + b_ref[...]

    return pl.pallas_call(
        body,
        grid=(_G,),
        in_specs=[
            pl.BlockSpec((2, _R, din), lambda i: (0, i, 0)),
            pl.BlockSpec((_R, din), lambda i: (i, 0)),
            pl.BlockSpec((_R, 1), lambda i: (i, 0)),
            pl.BlockSpec((1, dout), lambda i: (0, 0)),
            pl.BlockSpec((din, dout), lambda i: (0, 0)),
        ],
        out_specs=pl.BlockSpec((_R, dout), lambda i: (i, 0)),
        out_shape=jax.ShapeDtypeStruct((_N, dout), jnp.float32),
    )(p4, h3t, dinv, b, W)


def kernel(x, edge_index, W_e1, b_e1, W_e2, b_e2, W_d1, b_d1, W_d2, b_d2):
    ramp = jnp.arange(_EPAD, dtype=edge_index.dtype)
    pad_src = ramp % _N
    pad_dst = _N + ramp % (_NPAD - _N)
    srcf = jnp.concatenate([edge_index[0], pad_src])
    dstf = jnp.concatenate([edge_index[1], pad_dst])
    dst2d = dstf.reshape(2 * _CPC, _CH)
    onehot = jnp.zeros((_CH, 128), jnp.float32).at[:, 0].set(1.0)
    z128 = jnp.zeros((_WB, 128), jnp.float32)

    degtab = _sc_deg()(dst2d, onehot, z128)
    dinv, s1 = _tc_pre1(x, W_e1, degtab)
    p1 = _sc_agg(128)(srcf, dstf, s1, z128)
    s2 = _tc_mid(p1, s1, dinv, b_e1.reshape(1, -1), W_e2, "relu", pad_to=128)
    p2 = _sc_agg(128)(srcf, dstf, s2, z128)
    s3 = _tc_mid(p2, s2, dinv, b_e2.reshape(1, -1), W_d1, "norm", take=64)
    p3 = _sc_agg(128)(srcf, dstf, s3, z128)
    h3t = _tc_mid(p3, s3, dinv, b_d1.reshape(1, -1), None, "relu")
    p4 = _sc_agg(128)(srcf, dstf, h3t, z128)
    x_hat = _tc_post(p4, h3t, dinv, b_d2.reshape(1, -1), W_d2)
    return x_hat
